# baseline (speedup 1.0000x reference)
import jax
import jax.numpy as jnp
from jax.experimental import pallas as pl
from jax.experimental.pallas import tpu as pltpu

C = 128
E = 8
H = 8
D = 16
T = 32
NEG = -1e9
EPS = 1e-5


def _attn_body(x_ref, g_ref, b_ref, wq_ref, wk_ref, wv_ref, wp_ref, bp_ref, o_ref):
    xb = x_ref[...]
    BB = xb.shape[0]
    mu = jnp.mean(xb, axis=-1, keepdims=True)
    var = jnp.mean((xb - mu) ** 2, axis=-1, keepdims=True)
    xn = (xb - mu) / jnp.sqrt(var + EPS) * g_ref[...] + b_ref[...]
    x2 = xn.reshape(BB * T, C)
    q = jnp.dot(x2, wq_ref[...], preferred_element_type=jnp.float32).reshape(BB, T, H, D)
    k = jnp.dot(x2, wk_ref[...], preferred_element_type=jnp.float32).reshape(BB, T, H, D)
    v = jnp.dot(x2, wv_ref[...], preferred_element_type=jnp.float32).reshape(BB, T, H, D)
    row = jax.lax.broadcasted_iota(jnp.int32, (T, T), 0)
    col = jax.lax.broadcasted_iota(jnp.int32, (T, T), 1)
    causal = row >= col
    scale = C ** -0.5
    outs = []
    for h in range(H):
        qh = q[:, :, h, :]
        kh = k[:, :, h, :]
        vh = v[:, :, h, :]
        s = jax.lax.dot_general(qh, kh, (((2,), (2,)), ((0,), (0,))),
                                preferred_element_type=jnp.float32) * scale
        s = jnp.where(causal[None, :, :], s, NEG)
        m = jnp.max(s, axis=-1, keepdims=True)
        e = jnp.exp(s - m)
        p = e / jnp.sum(e, axis=-1, keepdims=True)
        oh = jax.lax.dot_general(p, vh, (((2,), (1,)), ((0,), (0,))),
                                 preferred_element_type=jnp.float32)
        outs.append(oh)
    o = jnp.concatenate(outs, axis=-1).reshape(BB * T, C)
    y = jnp.dot(o, wp_ref[...], preferred_element_type=jnp.float32) + bp_ref[...]
    o_ref[...] = xb + y.reshape(BB, T, C)


def _router_body(x1_ref, g_ref, b_ref, wcat_ref, bcat_ref, nz_ref, tri_ref,
                 h_ref, pos_ref, mask_ref, gate_ref, aux_ref, cnt_ref, acc_ref):
    i = pl.program_id(0)

    @pl.when(i == 0)
    def _():
        acc_ref[...] = jnp.zeros_like(acc_ref)

    xb = x1_ref[...]
    mu = jnp.mean(xb, axis=-1, keepdims=True)
    var = jnp.mean((xb - mu) ** 2, axis=-1, keepdims=True)
    hh = (xb - mu) / jnp.sqrt(var + EPS) * g_ref[...] + b_ref[...]
    h_ref[...] = hh

    lc = jnp.dot(hh, wcat_ref[...], preferred_element_type=jnp.float32) + bcat_ref[...]
    logits = lc[:, 0:E]
    nlog = lc[:, E:2 * E]
    sk = lc[:, 2 * E:2 * E + 1]
    softp = jnp.logaddexp(nlog, 0.0)
    noisy = logits + nz_ref[...] * softp

    r8 = jax.lax.broadcasted_iota(jnp.int32, (E, E), 0)
    c8 = jax.lax.broadcasted_iota(jnp.int32, (E, E), 1)
    l8 = (r8 <= c8).astype(jnp.float32)
    m1 = jnp.max(noisy, axis=-1, keepdims=True)
    eq1 = (noisy == m1).astype(jnp.float32)
    cs1 = jnp.dot(eq1, l8, preferred_element_type=jnp.float32)
    first = (eq1 > 0) & (cs1 == 1.0)
    noisy2 = jnp.where(first, -3e38, noisy)
    m2 = jnp.max(noisy2, axis=-1, keepdims=True)
    eq2 = (noisy2 == m2).astype(jnp.float32)
    cs2 = jnp.dot(eq2, l8, preferred_element_type=jnp.float32)
    second = (eq2 > 0) & (cs2 == 1.0)
    topk = first | second

    z = jnp.where(topk, jnp.exp(noisy - m1), 0.0)
    gate = z / jnp.sum(z, axis=-1, keepdims=True)
    gate_ref[...] = gate

    ns = (jax.nn.sigmoid(sk) <= 0.5).astype(jnp.float32)
    emask = topk.astype(jnp.float32) * ns
    mask_ref[...] = emask

    posb = jnp.dot(tri_ref[...], emask, preferred_element_type=jnp.float32)
    pos = posb + acc_ref[0:1, 0:E]
    pos_ref[...] = pos

    lane = jax.lax.broadcasted_iota(jnp.int32, emask.shape, 1)
    aux_ref[...] = jnp.where(lane == 0, ns, 0.0)

    new_cnt = acc_ref[0:1, 0:E] + jnp.sum(emask, axis=0, keepdims=True)
    new_ns = acc_ref[0:1, E:E + 1] + jnp.sum(ns, keepdims=True).reshape(1, 1)
    row = jnp.concatenate([new_cnt, new_ns, jnp.zeros((1, 16 - E - 1), jnp.float32)], axis=1)
    acc_ref[...] = row
    cnt_ref[...] = row


def _moe_dense_body(cnt_ref, h_ref, pos_ref, mask_ref, gate_ref, aux_ref, x1_ref,
                    w1_ref, b1_ref, w2_ref, b2_ref, o_ref, acc_ref):
    j = pl.program_id(1)
    ntok = cnt_ref[0, E]
    cap = jnp.floor(ntok * 2.0 / 8.0)
    pos = pos_ref[...]
    wmat = jnp.where(pos <= cap, mask_ref[...], 0.0) * gate_ref[...]
    onej = (jax.lax.broadcasted_iota(jnp.int32, (E, 1), 0) == j).astype(jnp.float32)
    wj = jnp.dot(wmat, onej, preferred_element_type=jnp.float32)
    hh = h_ref[...]
    t = jnp.maximum(jnp.dot(hh, w1_ref[0], preferred_element_type=jnp.float32)
                    + b1_ref[0], 0.0)
    eo = jnp.dot(t, w2_ref[0], preferred_element_type=jnp.float32) + b2_ref[0]

    @pl.when(j == 0)
    def _():
        acc_ref[...] = eo * wj

    @pl.when(j > 0)
    def _():
        acc_ref[...] += eo * wj

    @pl.when(j == E - 1)
    def _():
        ns = aux_ref[:, 0:1]
        o_ref[...] = x1_ref[...] + jnp.where(ns > 0.5, acc_ref[...], hh)


def kernel(x, ln1_g, ln1_b, Wq, Wk, Wv, Wp, bp, ln2_g, ln2_b, We, be, Wn, bn,
           Ws, bs, eW1, eb1, eW2, eb2):
    B = x.shape[0]
    N = B * T
    BB = 64
    BT = 512
    nb = B // BB
    nt = N // BT

    wq = Wq.transpose(1, 0, 2).reshape(C, C)
    wk = Wk.transpose(1, 0, 2).reshape(C, C)
    wv = Wv.transpose(1, 0, 2).reshape(C, C)
    g1 = ln1_g.reshape(1, C)
    b1 = ln1_b.reshape(1, C)
    bpr = bp.reshape(1, C)

    x1 = pl.pallas_call(
        _attn_body,
        grid=(nb,),
        in_specs=[
            pl.BlockSpec((BB, T, C), lambda i: (i, 0, 0)),
            pl.BlockSpec((1, C), lambda i: (0, 0)),
            pl.BlockSpec((1, C), lambda i: (0, 0)),
            pl.BlockSpec((C, C), lambda i: (0, 0)),
            pl.BlockSpec((C, C), lambda i: (0, 0)),
            pl.BlockSpec((C, C), lambda i: (0, 0)),
            pl.BlockSpec((C, C), lambda i: (0, 0)),
            pl.BlockSpec((1, C), lambda i: (0, 0)),
        ],
        out_specs=pl.BlockSpec((BB, T, C), lambda i: (i, 0, 0)),
        out_shape=jax.ShapeDtypeStruct((B, T, C), jnp.float32),
    )(x, g1, b1, wq, wk, wv, Wp, bpr)

    x1f = x1.reshape(N, C)
    wcat = jnp.zeros((C, 32), jnp.float32)
    wcat = wcat.at[:, 0:E].set(We).at[:, E:2 * E].set(Wn).at[:, 2 * E:2 * E + 1].set(Ws)
    bcat = jnp.zeros((1, 32), jnp.float32)
    bcat = bcat.at[0, 0:E].set(be).at[0, E:2 * E].set(bn).at[0, 2 * E:2 * E + 1].set(bs)
    nz = jax.random.normal(jax.random.key(42), (B, T, E), dtype=jnp.float32).reshape(N, E)
    tri = jnp.tril(jnp.ones((BT, BT), jnp.float32))
    g2 = ln2_g.reshape(1, C)
    b2 = ln2_b.reshape(1, C)

    h, pos, maskf, gate, aux, cnt = pl.pallas_call(
        _router_body,
        grid=(nt,),
        in_specs=[
            pl.BlockSpec((BT, C), lambda i: (i, 0)),
            pl.BlockSpec((1, C), lambda i: (0, 0)),
            pl.BlockSpec((1, C), lambda i: (0, 0)),
            pl.BlockSpec((C, 32), lambda i: (0, 0)),
            pl.BlockSpec((1, 32), lambda i: (0, 0)),
            pl.BlockSpec((BT, E), lambda i: (i, 0)),
            pl.BlockSpec((BT, BT), lambda i: (0, 0)),
        ],
        out_specs=[
            pl.BlockSpec((BT, C), lambda i: (i, 0)),
            pl.BlockSpec((BT, E), lambda i: (i, 0)),
            pl.BlockSpec((BT, E), lambda i: (i, 0)),
            pl.BlockSpec((BT, E), lambda i: (i, 0)),
            pl.BlockSpec((BT, E), lambda i: (i, 0)),
            pl.BlockSpec((1, 16), lambda i: (0, 0)),
        ],
        out_shape=[
            jax.ShapeDtypeStruct((N, C), jnp.float32),
            jax.ShapeDtypeStruct((N, E), jnp.float32),
            jax.ShapeDtypeStruct((N, E), jnp.float32),
            jax.ShapeDtypeStruct((N, E), jnp.float32),
            jax.ShapeDtypeStruct((N, E), jnp.float32),
            jax.ShapeDtypeStruct((1, 16), jnp.float32),
        ],
        scratch_shapes=[pltpu.VMEM((1, 16), jnp.float32)],
    )(x1f, g2, b2, wcat, bcat, nz, tri)

    eb1r = eb1.reshape(E, 1, 4 * C)
    eb2r = eb2.reshape(E, 1, C)

    out = pl.pallas_call(
        _moe_dense_body,
        grid=(nt, E),
        in_specs=[
            pl.BlockSpec((1, 16), lambda i, j: (0, 0)),
            pl.BlockSpec((BT, C), lambda i, j: (i, 0)),
            pl.BlockSpec((BT, E), lambda i, j: (i, 0)),
            pl.BlockSpec((BT, E), lambda i, j: (i, 0)),
            pl.BlockSpec((BT, E), lambda i, j: (i, 0)),
            pl.BlockSpec((BT, E), lambda i, j: (i, 0)),
            pl.BlockSpec((BT, C), lambda i, j: (i, 0)),
            pl.BlockSpec((1, C, 4 * C), lambda i, j: (j, 0, 0)),
            pl.BlockSpec((1, 1, 4 * C), lambda i, j: (j, 0, 0)),
            pl.BlockSpec((1, 4 * C, C), lambda i, j: (j, 0, 0)),
            pl.BlockSpec((1, 1, C), lambda i, j: (j, 0, 0)),
        ],
        out_specs=pl.BlockSpec((BT, C), lambda i, j: (i, 0)),
        out_shape=jax.ShapeDtypeStruct((N, C), jnp.float32),
        scratch_shapes=[pltpu.VMEM((BT, C), jnp.float32)],
    )(cnt, h, pos, maskf, gate, aux, x1f, eW1, eb1r, eW2, eb2r)

    return out.reshape(B, T, C)
